# baseline (speedup 1.0000x reference)
import functools

import jax
import jax.numpy as jnp
from jax import lax
from jax.experimental import pallas as pl
from jax.experimental.pallas import tpu as pltpu
from jax.experimental.pallas import tpu_sc as plsc


def _make_sc_broadcast(B, L, D, NC, NS):
    NW = NC * NS
    rows_per_w = B // NW
    row_words = L * D
    rows_per_sub = rows_per_w
    while rows_per_sub * NS * row_words * 4 > 7 * 1024 * 1024:
        rows_per_sub //= 2
    buf_rows = rows_per_sub * NS
    n_out_dma = rows_per_w // buf_rows if rows_per_w % buf_rows == 0 else 0

    mesh = plsc.VectorSubcoreMesh(core_axis_name="c", subcore_axis_name="s")

    @functools.partial(
        pl.kernel,
        mesh=mesh,
        out_type=jax.ShapeDtypeStruct((B, row_words), jnp.float32),
        scratch_types=[
            pltpu.VMEM_SHARED((buf_rows, row_words), jnp.float32),
            pltpu.SemaphoreType.DMA,
        ],
    )
    def k(table_hbm, out_hbm, spbuf, sem):
        cid = lax.axis_index("c")
        sid = lax.axis_index("s")
        wid = sid * NC + cid
        base = wid * rows_per_w
        stage = [
            pltpu.async_copy(table_hbm, spbuf.at[sid * rows_per_sub + r], sem)
            for r in range(rows_per_sub)
        ]
        for c in stage:
            c.wait()
        plsc.subcore_barrier()
        if n_out_dma:
            outs = [
                pltpu.async_copy(
                    spbuf, out_hbm.at[pl.ds(base + i * buf_rows, buf_rows)], sem
                )
                for i in range(n_out_dma)
            ]
            for c in outs:
                c.wait()

    return k


def kernel(x, pos_table):
    B, L = x.shape
    D = pos_table.shape[1]
    info = plsc.get_sparse_core_info()
    NC, NS = info.num_cores, info.num_subcores
    table_flat = pos_table[:L].reshape(L * D)
    k = _make_sc_broadcast(B, L, D, NC, NS)
    out = k(table_flat)
    return out.reshape(B, L, D)


# speedup vs baseline: 1.1369x; 1.1369x over previous
import functools

import jax
import jax.numpy as jnp
from jax import lax
from jax.experimental import pallas as pl
from jax.experimental.pallas import tpu as pltpu
from jax.experimental.pallas import tpu_sc as plsc


def _make_sc_broadcast(B, L, D, NC, NS):
    NW = NC * NS
    rows_per_w = B // NW
    row_words = L * D
    rep = 1
    for cand in range(min(rows_per_w, (120 * 1024) // row_words), 0, -1):
        if rows_per_w % cand == 0 and cand * row_words * 4 <= 480 * 1024:
            rep = cand
            break
    n_dma = rows_per_w // rep

    mesh = plsc.VectorSubcoreMesh(core_axis_name="c", subcore_axis_name="s")

    @functools.partial(
        pl.kernel,
        mesh=mesh,
        out_type=jax.ShapeDtypeStruct((B, row_words), jnp.float32),
        scratch_types=[
            pltpu.VMEM((rep, row_words), jnp.float32),
            pltpu.SemaphoreType.DMA,
        ],
    )
    def k(table_hbm, out_hbm, buf, sem):
        wid = lax.axis_index("s") * NC + lax.axis_index("c")
        base = wid * rows_per_w
        stage = [pltpu.async_copy(table_hbm, buf.at[r], sem) for r in range(rep)]
        for c in stage:
            c.wait()
        copies = [
            pltpu.async_copy(buf, out_hbm.at[pl.ds(base + i * rep, rep)], sem)
            for i in range(n_dma)
        ]
        for c in copies:
            c.wait()

    return k


def kernel(x, pos_table):
    B, L = x.shape
    D = pos_table.shape[1]
    info = plsc.get_sparse_core_info()
    NC, NS = info.num_cores, info.num_subcores
    table_flat = pos_table[:L].reshape(L * D)
    k = _make_sc_broadcast(B, L, D, NC, NS)
    out = k(table_flat)
    return out.reshape(B, L, D)
